# baseline (speedup 1.0000x reference)
import functools

import jax
import jax.numpy as jnp
import numpy as np
from jax import lax
from jax.experimental import pallas as pl
from jax.experimental.pallas import tpu as pltpu
from jax.experimental.pallas import tpu_sc as plsc

_PI = np.empty(128, np.int32)
for _g in range(4):
    for _i in range(16):
        _PI[32 * _g + _i] = 16 * _g + _i
        _PI[32 * _g + 16 + _i] = 64 + 16 * _g + _i

_N = 10000
_E = 320000
_D = 128
_H = 128
_ED = 16

_NC = 2
_NS = 16
_NW = _NC * _NS
_EPW = _E // _NW
_EK = 16
_NCHUNK = _EPW // _EK
_NBUF = 5
_NGRP = _NCHUNK // _NBUF
_LOOK = 3
_RPT = 624
_RTAIL = _N - _NS * _RPT


def _lin_body(x_ref, w_ref, b_ref, o_ref, *, act):
    y = jnp.dot(x_ref[...], w_ref[...], preferred_element_type=jnp.float32)
    y = y + b_ref[...]
    if act == "relu":
        y = jnp.maximum(y, 0.0)
    if act == "pack_i32":
        bu = lax.bitcast_convert_type(y, jnp.uint32)
        r = (bu + jnp.uint32(0x7FFF)
             + ((bu >> jnp.uint32(16)) & jnp.uint32(1))) >> jnp.uint32(16)
        w = (r[:, :_H // 2] & jnp.uint32(0xFFFF)) | (r[:, _H // 2:]
                                                     << jnp.uint32(16))
        y = lax.bitcast_convert_type(w, jnp.int32)
    o_ref[...] = y


def _linear(x, w, b, act, block_rows):
    rows, din = x.shape
    dout = w.shape[1]
    grid = rows // block_rows
    odt = jnp.int32 if act == "pack_i32" else jnp.float32
    odout = dout // 2 if act == "pack_i32" else dout
    return pl.pallas_call(
        functools.partial(_lin_body, act=act),
        grid=(grid,),
        in_specs=[
            pl.BlockSpec((block_rows, din), lambda i: (i, 0)),
            pl.BlockSpec((din, dout), lambda i: (0, 0)),
            pl.BlockSpec((1, dout), lambda i: (0, 0)),
        ],
        out_specs=pl.BlockSpec((block_rows, odout), lambda i: (i, 0)),
        out_shape=jax.ShapeDtypeStruct((rows, odout), odt),
    )(x, w, b.reshape(1, dout))


def _mlp_body(s_ref, h_ref, a_ref, w1_ref, b1_ref, w2_ref, b2_ref, o_ref):
    h = s_ref[0, 0] * h_ref[...] + a_ref[0] + a_ref[1]
    t = jnp.dot(h, w1_ref[...], preferred_element_type=jnp.float32) + b1_ref[...]
    t = jnp.maximum(t, 0.0)
    y = jnp.dot(t, w2_ref[...], preferred_element_type=jnp.float32) + b2_ref[...]
    o_ref[...] = jnp.where(y >= 0.0, y, 0.01 * y)


def _gine_mlp_leaky(scale, h, agg, w1, b1, w2, b2, block_rows=2000):
    grid = _N // block_rows
    return pl.pallas_call(
        _mlp_body,
        grid=(grid,),
        in_specs=[
            pl.BlockSpec(memory_space=pltpu.SMEM),
            pl.BlockSpec((block_rows, _H), lambda i: (i, 0)),
            pl.BlockSpec((_NC, block_rows, _H), lambda i: (0, i, 0)),
            pl.BlockSpec((_H, _H), lambda i: (0, 0)),
            pl.BlockSpec((1, _H), lambda i: (0, 0)),
            pl.BlockSpec((_H, _H), lambda i: (0, 0)),
            pl.BlockSpec((1, _H), lambda i: (0, 0)),
        ],
        out_specs=pl.BlockSpec((block_rows, _H), lambda i: (i, 0)),
        out_shape=jax.ShapeDtypeStruct((_N, _H), jnp.float32),
    )(scale, h, agg, w1, b1.reshape(1, _H), w2, b2.reshape(1, _H))


def _final_body(s_ref, h_ref, a_ref, w1_ref, b1_ref, w2_ref, b2_ref,
                pw1_ref, pb1_ref, pw2_ref, pb2_ref, o1_ref, o2_ref):
    h = s_ref[0, 0] * h_ref[...] + a_ref[0] + a_ref[1]
    t = jnp.dot(h, w1_ref[...], preferred_element_type=jnp.float32) + b1_ref[...]
    t = jnp.maximum(t, 0.0)
    oc = jnp.dot(t, w2_ref[...], preferred_element_type=jnp.float32) + b2_ref[...]
    o1_ref[...] = oc
    t2 = jnp.dot(oc, pw1_ref[...], preferred_element_type=jnp.float32) + pb1_ref[...]
    t2 = jnp.maximum(t2, 0.0)
    o2_ref[...] = jnp.dot(t2, pw2_ref[...], preferred_element_type=jnp.float32) + pb2_ref[...]


def _final_stage(scale, h, agg, w1, b1, w2, b2, pw1, pb1, pw2, pb2,
                 block_rows=2000):
    grid = _N // block_rows
    wspec = pl.BlockSpec((_H, _H), lambda i: (0, 0))
    bspec = pl.BlockSpec((1, _H), lambda i: (0, 0))
    return pl.pallas_call(
        _final_body,
        grid=(grid,),
        in_specs=[
            pl.BlockSpec(memory_space=pltpu.SMEM),
            pl.BlockSpec((block_rows, _H), lambda i: (i, 0)),
            pl.BlockSpec((_NC, block_rows, _H), lambda i: (0, i, 0)),
            wspec, bspec, wspec, bspec, wspec, bspec, wspec, bspec,
        ],
        out_specs=[
            pl.BlockSpec((block_rows, _H), lambda i: (i, 0)),
            pl.BlockSpec((block_rows, _H), lambda i: (i, 0)),
        ],
        out_shape=[
            jax.ShapeDtypeStruct((_N, _H), jnp.float32),
            jax.ShapeDtypeStruct((_N, _H), jnp.float32),
        ],
    )(scale, h, agg, w1, b1.reshape(1, _H), w2, b2.reshape(1, _H),
      pw1, pb1.reshape(1, _H), pw2, pb2.reshape(1, _H))


def _sc_agg_body(h_hbm, emb_hbm, src_hbm, dst_hbm, z_hbm, out_hbm,
                 src_all, dst_all, rows_v, emb_v, accum, *sems):
    gsems = sems[0:_NBUF]
    esems = sems[_NBUF:2 * _NBUF]
    ssems = sems[2 * _NBUF:3 * _NBUF]
    cid = lax.axis_index("c")
    sid = lax.axis_index("s")
    wid = sid * _NC + cid

    row0 = pl.multiple_of(sid * _RPT, 8)
    ebase = pl.multiple_of(wid * _EPW, 8)

    pltpu.sync_copy(z_hbm.at[pl.ds(row0, _RPT)], accum.at[pl.ds(row0, _RPT)])

    @pl.when(sid == _NS - 1)
    def _zero_tail():
        pltpu.sync_copy(z_hbm.at[pl.ds(_NS * _RPT, _RTAIL)],
                        accum.at[pl.ds(_NS * _RPT, _RTAIL)])

    pltpu.sync_copy(src_hbm.at[pl.ds(ebase, _EPW)], src_all)
    pltpu.sync_copy(dst_hbm.at[pl.ds(ebase, _EPW)], dst_all)
    plsc.subcore_barrier()

    def issue(i, b):
        idx = src_all.at[pl.ds(pl.multiple_of(i * _EK, 8), _EK)]
        pltpu.async_copy(h_hbm.at[idx], rows_v.at[b], gsems[b])
        eoff = pl.multiple_of(wid * _EPW + i * _EK, 8)
        pltpu.async_copy(emb_hbm.at[pl.ds(eoff, _EK)], emb_v[b], esems[b])

    def wait_in(b):
        pltpu.make_async_copy(h_hbm.at[src_all.at[pl.ds(0, _EK)]],
                              rows_v.at[b], gsems[b]).wait()
        pltpu.make_async_copy(emb_hbm.at[pl.ds(0, _EK)], emb_v[b],
                              esems[b]).wait()

    def didx(i):
        return dst_all.at[pl.ds(pl.multiple_of(i * _EK, 8), _EK)]

    def wait_s(b):
        pltpu.make_async_copy(rows_v.at[b], accum.at[didx(0)],
                              ssems[b]).wait()

    for b in range(_LOOK):
        issue(b, b)

    def group(k, carry):
        for b in range(_NBUF):
            i = k * _NBUF + b
            wait_in(b)

            def edge(j, c, _b=b):
                for g in range(_H // 32):
                    ei = emb_v[_b][j, pl.ds(g * 16, 16)]
                    lof = plsc.bitcast(ei << 16, jnp.float32)
                    hif = plsc.bitcast(ei & jnp.int32(-65536), jnp.float32)
                    sl_lo = pl.ds(g * 32, 16)
                    sl_hi = pl.ds(g * 32 + 16, 16)
                    rows_v[_b, j, sl_lo] = jnp.maximum(
                        rows_v[_b, j, sl_lo] + lof, 0.0)
                    rows_v[_b, j, sl_hi] = jnp.maximum(
                        rows_v[_b, j, sl_hi] + hif, 0.0)
                return c

            lax.fori_loop(0, _EK, edge, 0)
            pltpu.async_copy(rows_v.at[b], accum.at[didx(i)],
                             ssems[b], add=True)

            bj = (b + _LOOK) % _NBUF
            if b < _NBUF - _LOOK:
                @pl.when(k > 0)
                def _drain(_bj=bj):
                    wait_s(_bj)

                issue(i + _LOOK, bj)
            else:
                wait_s(bj)

                @pl.when(k < _NGRP - 1)
                def _refill(_i=i, _bj=bj):
                    issue(_i + _LOOK, _bj)
        return carry

    lax.fori_loop(0, _NGRP, group, 0)
    for b in range(_LOOK, _NBUF):
        wait_s(b)
    plsc.subcore_barrier()

    pltpu.sync_copy(accum.at[pl.ds(row0, _RPT)],
                    out_hbm.at[cid, pl.ds(row0, _RPT)])

    @pl.when(sid == _NS - 1)
    def _out_tail():
        pltpu.sync_copy(accum.at[pl.ds(_NS * _RPT, _RTAIL)],
                        out_hbm.at[cid, pl.ds(_NS * _RPT, _RTAIL)])


def _sc_agg(h, emb, src, dst):
    mesh = plsc.VectorSubcoreMesh(core_axis_name="c", subcore_axis_name="s",
                                  num_cores=_NC, num_subcores=_NS)
    zeros = jnp.zeros((_N, _H), jnp.float32)
    run = pl.kernel(
        _sc_agg_body,
        out_type=jax.ShapeDtypeStruct((_NC, _N, _H), jnp.float32),
        mesh=mesh,
        compiler_params=pltpu.CompilerParams(needs_layout_passes=False),
        scratch_types=[
            pltpu.VMEM((_EPW,), jnp.int32),
            pltpu.VMEM((_EPW,), jnp.int32),
            pltpu.VMEM((_NBUF, _EK, _H), jnp.float32),
            [pltpu.VMEM((_EK, _H // 2), jnp.int32) for _ in range(_NBUF)],
            pltpu.VMEM_SHARED((_N, _H), jnp.float32),
        ] + [pltpu.SemaphoreType.DMA] * (3 * _NBUF),
    )
    return run(h, emb, src, dst, zeros)


def kernel(x_customer, x_article, edge_attr_buys, edge_attr_rev, params,
           edge_index_buys, edge_index_rev):
    p = params
    g1 = p["c1_buys"]
    g2 = p["c2_rev"]

    src1 = edge_index_buys[0].astype(jnp.int32)
    dst1 = edge_index_buys[1].astype(jnp.int32)
    src2 = edge_index_rev[0].astype(jnp.int32)
    dst2 = edge_index_rev[1].astype(jnp.int32)

    pi = jnp.asarray(_PI)
    h_c = _linear(x_customer, p["lin_c_W"][:, pi], p["lin_c_b"][pi],
                  "relu", 2000)
    emb1 = _linear(edge_attr_buys, g1["We"], g1["be"], "pack_i32", 4000)
    emb2 = _linear(edge_attr_rev, g2["We"], g2["be"], "pack_i32", 4000)

    h_c, emb2 = lax.optimization_barrier((h_c, emb2))

    agg1 = _sc_agg(h_c, emb1, src1, dst1)
    s1 = (1.0 + g1["eps"]).reshape(1, 1)
    h_a = _gine_mlp_leaky(s1, h_c, agg1, g1["W1"][pi, :], g1["b1"],
                          g1["W2"][:, pi], g1["b2"][pi])

    agg2 = _sc_agg(h_a, emb2, src2, dst2)
    s2 = (1.0 + g2["eps"]).reshape(1, 1)
    out_c2, proj_c = _final_stage(
        s2, h_a, agg2, g2["W1"][pi, :], g2["b1"], g2["W2"], g2["b2"],
        p["proj_c_W1"], p["proj_c_b1"], p["proj_c_W2"], p["proj_c_b2"])
    return out_c2, proj_c


# speedup vs baseline: 1.2337x; 1.2337x over previous
import functools

import jax
import jax.numpy as jnp
import numpy as np
from jax import lax
from jax.experimental import pallas as pl
from jax.experimental.pallas import tpu as pltpu
from jax.experimental.pallas import tpu_sc as plsc

_PI = np.empty(128, np.int32)
for _g in range(4):
    for _i in range(16):
        _PI[32 * _g + _i] = 16 * _g + _i
        _PI[32 * _g + 16 + _i] = 64 + 16 * _g + _i

_N = 10000
_E = 320000
_D = 128
_H = 128
_ED = 16

_NC = 2
_NS = 16
_NW = _NC * _NS
_EPW = _E // _NW
_EK = 16
_NCHUNK = _EPW // _EK
_NBUF = 5
_NGRP = _NCHUNK // _NBUF
_LOOK = 4
_RPT = 624
_RTAIL = _N - _NS * _RPT


def _lin_body(x_ref, w_ref, b_ref, o_ref, *, act):
    y = jnp.dot(x_ref[...], w_ref[...], preferred_element_type=jnp.float32)
    y = y + b_ref[...]
    if act == "relu":
        y = jnp.maximum(y, 0.0)
    if act == "pack_i32":
        bu = lax.bitcast_convert_type(y, jnp.uint32)
        r = (bu + jnp.uint32(0x7FFF)
             + ((bu >> jnp.uint32(16)) & jnp.uint32(1))) >> jnp.uint32(16)
        w = (r[:, :_H // 2] & jnp.uint32(0xFFFF)) | (r[:, _H // 2:]
                                                     << jnp.uint32(16))
        y = lax.bitcast_convert_type(w, jnp.int32)
    o_ref[...] = y


def _linear(x, w, b, act, block_rows):
    rows, din = x.shape
    dout = w.shape[1]
    grid = rows // block_rows
    odt = jnp.int32 if act == "pack_i32" else jnp.float32
    odout = dout // 2 if act == "pack_i32" else dout
    return pl.pallas_call(
        functools.partial(_lin_body, act=act),
        grid=(grid,),
        in_specs=[
            pl.BlockSpec((block_rows, din), lambda i: (i, 0)),
            pl.BlockSpec((din, dout), lambda i: (0, 0)),
            pl.BlockSpec((1, dout), lambda i: (0, 0)),
        ],
        out_specs=pl.BlockSpec((block_rows, odout), lambda i: (i, 0)),
        out_shape=jax.ShapeDtypeStruct((rows, odout), odt),
    )(x, w, b.reshape(1, dout))


def _mlp_body(s_ref, h_ref, a_ref, w1_ref, b1_ref, w2_ref, b2_ref, o_ref):
    h = s_ref[0, 0] * h_ref[...] + a_ref[0] + a_ref[1]
    t = jnp.dot(h, w1_ref[...], preferred_element_type=jnp.float32) + b1_ref[...]
    t = jnp.maximum(t, 0.0)
    y = jnp.dot(t, w2_ref[...], preferred_element_type=jnp.float32) + b2_ref[...]
    o_ref[...] = jnp.where(y >= 0.0, y, 0.01 * y)


def _gine_mlp_leaky(scale, h, agg, w1, b1, w2, b2, block_rows=2000):
    grid = _N // block_rows
    return pl.pallas_call(
        _mlp_body,
        grid=(grid,),
        in_specs=[
            pl.BlockSpec(memory_space=pltpu.SMEM),
            pl.BlockSpec((block_rows, _H), lambda i: (i, 0)),
            pl.BlockSpec((_NC, block_rows, _H), lambda i: (0, i, 0)),
            pl.BlockSpec((_H, _H), lambda i: (0, 0)),
            pl.BlockSpec((1, _H), lambda i: (0, 0)),
            pl.BlockSpec((_H, _H), lambda i: (0, 0)),
            pl.BlockSpec((1, _H), lambda i: (0, 0)),
        ],
        out_specs=pl.BlockSpec((block_rows, _H), lambda i: (i, 0)),
        out_shape=jax.ShapeDtypeStruct((_N, _H), jnp.float32),
    )(scale, h, agg, w1, b1.reshape(1, _H), w2, b2.reshape(1, _H))


def _final_body(s_ref, h_ref, a_ref, w1_ref, b1_ref, w2_ref, b2_ref,
                pw1_ref, pb1_ref, pw2_ref, pb2_ref, o1_ref, o2_ref):
    h = s_ref[0, 0] * h_ref[...] + a_ref[0] + a_ref[1]
    t = jnp.dot(h, w1_ref[...], preferred_element_type=jnp.float32) + b1_ref[...]
    t = jnp.maximum(t, 0.0)
    oc = jnp.dot(t, w2_ref[...], preferred_element_type=jnp.float32) + b2_ref[...]
    o1_ref[...] = oc
    t2 = jnp.dot(oc, pw1_ref[...], preferred_element_type=jnp.float32) + pb1_ref[...]
    t2 = jnp.maximum(t2, 0.0)
    o2_ref[...] = jnp.dot(t2, pw2_ref[...], preferred_element_type=jnp.float32) + pb2_ref[...]


def _final_stage(scale, h, agg, w1, b1, w2, b2, pw1, pb1, pw2, pb2,
                 block_rows=2000):
    grid = _N // block_rows
    wspec = pl.BlockSpec((_H, _H), lambda i: (0, 0))
    bspec = pl.BlockSpec((1, _H), lambda i: (0, 0))
    return pl.pallas_call(
        _final_body,
        grid=(grid,),
        in_specs=[
            pl.BlockSpec(memory_space=pltpu.SMEM),
            pl.BlockSpec((block_rows, _H), lambda i: (i, 0)),
            pl.BlockSpec((_NC, block_rows, _H), lambda i: (0, i, 0)),
            wspec, bspec, wspec, bspec, wspec, bspec, wspec, bspec,
        ],
        out_specs=[
            pl.BlockSpec((block_rows, _H), lambda i: (i, 0)),
            pl.BlockSpec((block_rows, _H), lambda i: (i, 0)),
        ],
        out_shape=[
            jax.ShapeDtypeStruct((_N, _H), jnp.float32),
            jax.ShapeDtypeStruct((_N, _H), jnp.float32),
        ],
    )(scale, h, agg, w1, b1.reshape(1, _H), w2, b2.reshape(1, _H),
      pw1, pb1.reshape(1, _H), pw2, pb2.reshape(1, _H))


def _sc_agg_body(h_hbm, emb_hbm, src_hbm, dst_hbm, z_hbm, out_hbm,
                 src_all, dst_all, rows_v, emb_v, accum, *sems):
    gsems = sems[0:_NBUF]
    esems = sems[_NBUF:2 * _NBUF]
    ssems = sems[2 * _NBUF:3 * _NBUF]
    cid = lax.axis_index("c")
    sid = lax.axis_index("s")
    wid = sid * _NC + cid

    row0 = pl.multiple_of(sid * _RPT, 8)
    ebase = pl.multiple_of(wid * _EPW, 8)

    pltpu.sync_copy(z_hbm.at[pl.ds(row0, _RPT)], accum.at[pl.ds(row0, _RPT)])

    @pl.when(sid == _NS - 1)
    def _zero_tail():
        pltpu.sync_copy(z_hbm.at[pl.ds(_NS * _RPT, _RTAIL)],
                        accum.at[pl.ds(_NS * _RPT, _RTAIL)])

    pltpu.sync_copy(src_hbm.at[pl.ds(ebase, _EPW)], src_all)
    pltpu.sync_copy(dst_hbm.at[pl.ds(ebase, _EPW)], dst_all)
    plsc.subcore_barrier()

    def issue(i, b):
        idx = src_all.at[pl.ds(pl.multiple_of(i * _EK, 8), _EK)]
        pltpu.async_copy(h_hbm.at[idx], rows_v.at[b], gsems[b])
        eoff = pl.multiple_of(wid * _EPW + i * _EK, 8)
        pltpu.async_copy(emb_hbm.at[pl.ds(eoff, _EK)], emb_v[b], esems[b])

    def wait_in(b):
        pltpu.make_async_copy(h_hbm.at[src_all.at[pl.ds(0, _EK)]],
                              rows_v.at[b], gsems[b]).wait()
        pltpu.make_async_copy(emb_hbm.at[pl.ds(0, _EK)], emb_v[b],
                              esems[b]).wait()

    def didx(i):
        return dst_all.at[pl.ds(pl.multiple_of(i * _EK, 8), _EK)]

    def wait_s(b):
        pltpu.make_async_copy(rows_v.at[b], accum.at[didx(0)],
                              ssems[b]).wait()

    for b in range(_LOOK):
        issue(b, b)

    def group(k, carry):
        for b in range(_NBUF):
            i = k * _NBUF + b
            wait_in(b)

            def edge(j, c, _b=b):
                for g in range(_H // 32):
                    ei = emb_v[_b][j, pl.ds(g * 16, 16)]
                    lof = plsc.bitcast(ei << 16, jnp.float32)
                    hif = plsc.bitcast(ei & jnp.int32(-65536), jnp.float32)
                    sl_lo = pl.ds(g * 32, 16)
                    sl_hi = pl.ds(g * 32 + 16, 16)
                    rows_v[_b, j, sl_lo] = jnp.maximum(
                        rows_v[_b, j, sl_lo] + lof, 0.0)
                    rows_v[_b, j, sl_hi] = jnp.maximum(
                        rows_v[_b, j, sl_hi] + hif, 0.0)
                return c

            lax.fori_loop(0, _EK, edge, 0)
            pltpu.async_copy(rows_v.at[b], accum.at[didx(i)],
                             ssems[b], add=True)

            bj = (b + _LOOK) % _NBUF
            if b < _NBUF - _LOOK:
                @pl.when(k > 0)
                def _drain(_bj=bj):
                    wait_s(_bj)

                issue(i + _LOOK, bj)
            else:
                wait_s(bj)

                @pl.when(k < _NGRP - 1)
                def _refill(_i=i, _bj=bj):
                    issue(_i + _LOOK, _bj)
        return carry

    lax.fori_loop(0, _NGRP, group, 0)
    for b in range(_LOOK, _NBUF):
        wait_s(b)
    plsc.subcore_barrier()

    pltpu.sync_copy(accum.at[pl.ds(row0, _RPT)],
                    out_hbm.at[cid, pl.ds(row0, _RPT)])

    @pl.when(sid == _NS - 1)
    def _out_tail():
        pltpu.sync_copy(accum.at[pl.ds(_NS * _RPT, _RTAIL)],
                        out_hbm.at[cid, pl.ds(_NS * _RPT, _RTAIL)])


def _sc_agg(h, emb, src, dst):
    mesh = plsc.VectorSubcoreMesh(core_axis_name="c", subcore_axis_name="s",
                                  num_cores=_NC, num_subcores=_NS)
    zeros = jnp.zeros((_N, _H), jnp.float32)
    run = pl.kernel(
        _sc_agg_body,
        out_type=jax.ShapeDtypeStruct((_NC, _N, _H), jnp.float32),
        mesh=mesh,
        compiler_params=pltpu.CompilerParams(needs_layout_passes=False),
        scratch_types=[
            pltpu.VMEM((_EPW,), jnp.int32),
            pltpu.VMEM((_EPW,), jnp.int32),
            pltpu.VMEM((_NBUF, _EK, _H), jnp.float32),
            [pltpu.VMEM((_EK, _H // 2), jnp.int32) for _ in range(_NBUF)],
            pltpu.VMEM_SHARED((_N, _H), jnp.float32),
        ] + [pltpu.SemaphoreType.DMA] * (3 * _NBUF),
    )
    return run(h, emb, src, dst, zeros)


def kernel(x_customer, x_article, edge_attr_buys, edge_attr_rev, params,
           edge_index_buys, edge_index_rev):
    p = params
    g1 = p["c1_buys"]
    g2 = p["c2_rev"]

    src1 = edge_index_buys[0].astype(jnp.int32)
    dst1 = edge_index_buys[1].astype(jnp.int32)
    src2 = edge_index_rev[0].astype(jnp.int32)
    dst2 = edge_index_rev[1].astype(jnp.int32)

    pi = jnp.asarray(_PI)
    h_c = _linear(x_customer, p["lin_c_W"][:, pi], p["lin_c_b"][pi],
                  "relu", 2000)
    emb1 = _linear(edge_attr_buys, g1["We"], g1["be"], "pack_i32", 4000)
    emb2 = _linear(edge_attr_rev, g2["We"], g2["be"], "pack_i32", 4000)

    agg1 = _sc_agg(h_c, emb1, src1, dst1)
    s1 = (1.0 + g1["eps"]).reshape(1, 1)
    h_a = _gine_mlp_leaky(s1, h_c, agg1, g1["W1"][pi, :], g1["b1"],
                          g1["W2"][:, pi], g1["b2"][pi])

    agg2 = _sc_agg(h_a, emb2, src2, dst2)
    s2 = (1.0 + g2["eps"]).reshape(1, 1)
    out_c2, proj_c = _final_stage(
        s2, h_a, agg2, g2["W1"][pi, :], g2["b1"], g2["W2"], g2["b2"],
        p["proj_c_W1"], p["proj_c_b1"], p["proj_c_W2"], p["proj_c_b2"])
    return out_c2, proj_c
